# baseline (speedup 1.0000x reference)
import functools

import jax
import jax.numpy as jnp
from jax import lax
from jax.experimental import pallas as pl
from jax.experimental.pallas import tpu as pltpu
from jax.experimental.pallas import tpu_sc as plsc

B = 16384
K = 32
KB = 128
V = 1000000
VB = V * K // KB
L = 16
NC, NS = 2, 16
NW = NC * NS
BPW = B // NW
CH = 128
NCH = BPW // CH


def _bprmf_sc(users, items, gu_tab, gi_tab):
  mesh = plsc.VectorSubcoreMesh(core_axis_name="c", subcore_axis_name="s")
  out_type = (
      jax.ShapeDtypeStruct((B,), jnp.float32),
      jax.ShapeDtypeStruct((B * K,), jnp.float32),
      jax.ShapeDtypeStruct((B * K,), jnp.float32),
  )
  cp = pltpu.CompilerParams(needs_layout_passes=False,
                            use_tc_tiling_on_sc=True)

  @functools.partial(
      pl.kernel,
      mesh=mesh,
      out_type=out_type,
      compiler_params=cp,
      scratch_types=[
          pltpu.VMEM((BPW,), jnp.int32),
          pltpu.VMEM((BPW,), jnp.int32),
          pltpu.VMEM((BPW,), jnp.int32),
          pltpu.VMEM((BPW,), jnp.int32),
          pltpu.VMEM((2, CH, KB), jnp.float32),
          pltpu.VMEM((2, CH, KB), jnp.float32),
          pltpu.VMEM((BPW * K,), jnp.float32),
          pltpu.VMEM((BPW * K,), jnp.float32),
          pltpu.VMEM((BPW,), jnp.float32),
          pltpu.SemaphoreType.DMA,
          pltpu.SemaphoreType.DMA,
          pltpu.SemaphoreType.DMA,
          pltpu.SemaphoreType.DMA,
      ],
  )
  def k(users_hbm, items_hbm, gu_hbm, gi_hbm, xui_hbm, guo_hbm, gio_hbm,
        uidx_v, iidx_v, ublk_v, iblk_v, guB, giB, gu_v, gi_v, xui_v,
        sem_u0, sem_u1, sem_i0, sem_i1):
    sem_u = (sem_u0, sem_u1)
    sem_i = (sem_i0, sem_i1)
    wid = lax.axis_index("s") * NC + lax.axis_index("c")
    base = wid * BPW

    pltpu.sync_copy(users_hbm.at[pl.ds(base, BPW)], uidx_v)
    pltpu.sync_copy(items_hbm.at[pl.ds(base, BPW)], iidx_v)

    @pl.loop(0, BPW // L)
    def _(g):
      sl = pl.ds(g * L, L)
      ublk_v[sl] = uidx_v[sl] >> 2
      iblk_v[sl] = iidx_v[sl] >> 2

    def fire(c):
      b = c % 2
      sl = pl.ds(c * CH, CH)
      return (
          pltpu.async_copy(gu_hbm.at[ublk_v.at[sl]], guB.at[b], sem_u[b]),
          pltpu.async_copy(gi_hbm.at[iblk_v.at[sl]], giB.at[b], sem_i[b]),
      )

    pending = fire(0)
    for c in range(NCH):
      nxt = fire(c + 1) if c + 1 < NCH else None
      for cp_ in pending:
        cp_.wait()
      b = c % 2

      @pl.loop(0, CH // L)
      def _(gg, c=c, b=b):
        j0 = c * CH + gg * L
        lrow = lax.iota(jnp.int32, L) + gg * L
        u16 = uidx_v[pl.ds(j0, L)]
        i16 = iidx_v[pl.ds(j0, L)]
        cbu = (u16 & 3) << 5
        cbi = (i16 & 3) << 5
        jflat = (lrow + c * CH) << 5
        acc = jnp.zeros((L,), jnp.float32)
        for kk in range(K):
          uk = plsc.load_gather(guB.at[b], [lrow, cbu + kk])
          ik = plsc.load_gather(giB.at[b], [lrow, cbi + kk])
          acc = acc + uk * ik
          plsc.store_scatter(gu_v, [jflat + kk], uk)
          plsc.store_scatter(gi_v, [jflat + kk], ik)
        xui_v[pl.ds(j0, L)] = acc

      pending = nxt

    pltpu.sync_copy(xui_v, xui_hbm.at[pl.ds(base, BPW)])
    pltpu.sync_copy(gu_v, guo_hbm.at[pl.ds(base * K, BPW * K)])
    pltpu.sync_copy(gi_v, gio_hbm.at[pl.ds(base * K, BPW * K)])

  return k(users, items, gu_tab, gi_tab)


def _blocked(tab):
  return jnp.concatenate([tab[a::4] for a in range(4)], axis=1)


def kernel(users, items, Gu, Gi):
  users = users.astype(jnp.int32)
  items = items.astype(jnp.int32)
  xui, guo, gio = _bprmf_sc(users, items, _blocked(Gu), _blocked(Gi))
  return (xui, guo.reshape(B, K), gio.reshape(B, K))


# speedup vs baseline: 15.6673x; 15.6673x over previous
import functools

import jax
import jax.numpy as jnp
from jax import lax
from jax.experimental import pallas as pl
from jax.experimental.pallas import tpu as pltpu
from jax.experimental.pallas import tpu_sc as plsc

B = 16384
K = 32
KB = 128
V = 1000000
L = 16
NC, NS = 2, 16
NW = NC * NS
BPW = B // NW
CH = 128
NCH = BPW // CH

CW = 8192
SEG = CW // 4
NBLK = -(-V // CW)
VB = NBLK * SEG


def _bprmf_sc(users, items, gu_tab, gi_tab):
  mesh = plsc.VectorSubcoreMesh(core_axis_name="c", subcore_axis_name="s")
  out_type = (
      jax.ShapeDtypeStruct((B,), jnp.float32),
      jax.ShapeDtypeStruct((B * K,), jnp.float32),
      jax.ShapeDtypeStruct((B * K,), jnp.float32),
  )
  cp = pltpu.CompilerParams(needs_layout_passes=False,
                            use_tc_tiling_on_sc=True)

  @functools.partial(
      pl.kernel,
      mesh=mesh,
      out_type=out_type,
      compiler_params=cp,
      scratch_types=[
          pltpu.VMEM((BPW,), jnp.int32),
          pltpu.VMEM((BPW,), jnp.int32),
          pltpu.VMEM((BPW,), jnp.int32),
          pltpu.VMEM((BPW,), jnp.int32),
          pltpu.VMEM((2, CH, KB), jnp.float32),
          pltpu.VMEM((2, CH, KB), jnp.float32),
          pltpu.VMEM((BPW * K,), jnp.float32),
          pltpu.VMEM((BPW * K,), jnp.float32),
          pltpu.VMEM((BPW,), jnp.float32),
          pltpu.SemaphoreType.DMA,
          pltpu.SemaphoreType.DMA,
          pltpu.SemaphoreType.DMA,
          pltpu.SemaphoreType.DMA,
      ],
  )
  def k(users_hbm, items_hbm, gu_hbm, gi_hbm, xui_hbm, guo_hbm, gio_hbm,
        uidx_v, iidx_v, ublk_v, iblk_v, guB, giB, gu_v, gi_v, xui_v,
        sem_u0, sem_u1, sem_i0, sem_i1):
    sem_u = (sem_u0, sem_u1)
    sem_i = (sem_i0, sem_i1)
    wid = lax.axis_index("s") * NC + lax.axis_index("c")
    base = wid * BPW

    pltpu.sync_copy(users_hbm.at[pl.ds(base, BPW)], uidx_v)
    pltpu.sync_copy(items_hbm.at[pl.ds(base, BPW)], iidx_v)

    @pl.loop(0, BPW // L)
    def _(g):
      sl = pl.ds(g * L, L)
      u = uidx_v[sl]
      i = iidx_v[sl]
      ublk_v[sl] = ((u >> 13) << 11) | (u & (SEG - 1))
      iblk_v[sl] = ((i >> 13) << 11) | (i & (SEG - 1))

    def fire(c):
      b = c % 2
      sl = pl.ds(c * CH, CH)
      return (
          pltpu.async_copy(gu_hbm.at[ublk_v.at[sl]], guB.at[b], sem_u[b]),
          pltpu.async_copy(gi_hbm.at[iblk_v.at[sl]], giB.at[b], sem_i[b]),
      )

    pending = fire(0)
    for c in range(NCH):
      nxt = fire(c + 1) if c + 1 < NCH else None
      for cp_ in pending:
        cp_.wait()
      b = c % 2

      @pl.loop(0, CH // L)
      def _(gg, c=c, b=b):
        j0 = c * CH + gg * L
        lrow = lax.iota(jnp.int32, L) + gg * L
        u16 = uidx_v[pl.ds(j0, L)]
        i16 = iidx_v[pl.ds(j0, L)]
        cbu = ((u16 >> 11) & 3) << 5
        cbi = ((i16 >> 11) & 3) << 5
        jflat = (lrow + c * CH) << 5
        acc = jnp.zeros((L,), jnp.float32)
        for kk in range(K):
          uk = plsc.load_gather(guB.at[b], [lrow, cbu + kk])
          ik = plsc.load_gather(giB.at[b], [lrow, cbi + kk])
          acc = acc + uk * ik
          plsc.store_scatter(gu_v, [jflat + kk], uk)
          plsc.store_scatter(gi_v, [jflat + kk], ik)
        xui_v[pl.ds(j0, L)] = acc

      pending = nxt

    pltpu.sync_copy(xui_v, xui_hbm.at[pl.ds(base, BPW)])
    pltpu.sync_copy(gu_v, guo_hbm.at[pl.ds(base * K, BPW * K)])
    pltpu.sync_copy(gi_v, gio_hbm.at[pl.ds(base * K, BPW * K)])

  return k(users, items, gu_tab, gi_tab)


def _transpose_block(xt_ref, out_ref):
  eye = (lax.broadcasted_iota(jnp.int32, (K, K), 0)
         == lax.broadcasted_iota(jnp.int32, (K, K), 1)).astype(jnp.float32)
  t = lax.dot_general(xt_ref[...], eye, (((0,), (0,)), ((), ())),
                      preferred_element_type=jnp.float32)
  out_ref[...] = jnp.concatenate(
      [t[r * SEG:(r + 1) * SEG, :] for r in range(4)], axis=1)


def _blocked(tab_t):
  return pl.pallas_call(
      _transpose_block,
      grid=(NBLK,),
      in_specs=[pl.BlockSpec((K, CW), lambda c: (0, c))],
      out_specs=pl.BlockSpec((SEG, KB), lambda c: (c, 0)),
      out_shape=jax.ShapeDtypeStruct((VB, KB), jnp.float32),
  )(tab_t)


def kernel(users, items, Gu, Gi):
  users = users.astype(jnp.int32)
  items = items.astype(jnp.int32)
  xui, guo, gio = _bprmf_sc(users, items, _blocked(Gu.T), _blocked(Gi.T))
  return (xui, guo.reshape(B, K), gio.reshape(B, K))


# speedup vs baseline: 15.7308x; 1.0041x over previous
import functools

import jax
import jax.numpy as jnp
from jax import lax
from jax.experimental import pallas as pl
from jax.experimental.pallas import tpu as pltpu
from jax.experimental.pallas import tpu_sc as plsc

B = 16384
K = 32
KB = 128
V = 1000000
L = 16
NC, NS = 2, 16
NW = NC * NS
BPW = B // NW
CH = 128
NCH = BPW // CH

CW = 8192
SEG = CW // 4
NBLK = -(-V // CW)
VB = NBLK * SEG


def _bprmf_sc(users, items, gu_tab, gi_tab):
  mesh = plsc.VectorSubcoreMesh(core_axis_name="c", subcore_axis_name="s")
  out_type = (
      jax.ShapeDtypeStruct((B,), jnp.float32),
      jax.ShapeDtypeStruct((B * K,), jnp.float32),
      jax.ShapeDtypeStruct((B * K,), jnp.float32),
  )
  cp = pltpu.CompilerParams(needs_layout_passes=False,
                            use_tc_tiling_on_sc=True)

  @functools.partial(
      pl.kernel,
      mesh=mesh,
      out_type=out_type,
      compiler_params=cp,
      scratch_types=[
          pltpu.VMEM((BPW,), jnp.int32),
          pltpu.VMEM((BPW,), jnp.int32),
          pltpu.VMEM((BPW,), jnp.int32),
          pltpu.VMEM((BPW,), jnp.int32),
          pltpu.VMEM((2, CH, KB), jnp.float32),
          pltpu.VMEM((2, CH, KB), jnp.float32),
          pltpu.VMEM((BPW * K,), jnp.float32),
          pltpu.VMEM((BPW * K,), jnp.float32),
          pltpu.VMEM((BPW,), jnp.float32),
          pltpu.SemaphoreType.DMA,
          pltpu.SemaphoreType.DMA,
          pltpu.SemaphoreType.DMA,
          pltpu.SemaphoreType.DMA,
      ],
  )
  def k(users_hbm, items_hbm, gu_hbm, gi_hbm, xui_hbm, guo_hbm, gio_hbm,
        uidx_v, iidx_v, ublk_v, iblk_v, guB, giB, gu_v, gi_v, xui_v,
        sem_u0, sem_u1, sem_i0, sem_i1):
    sem_u = (sem_u0, sem_u1)
    sem_i = (sem_i0, sem_i1)
    wid = lax.axis_index("s") * NC + lax.axis_index("c")
    base = wid * BPW

    pltpu.sync_copy(users_hbm.at[pl.ds(base, BPW)], uidx_v)
    pltpu.sync_copy(items_hbm.at[pl.ds(base, BPW)], iidx_v)

    @pl.loop(0, BPW // L)
    def _(g):
      sl = pl.ds(g * L, L)
      u = uidx_v[sl]
      i = iidx_v[sl]
      ublk_v[sl] = ((u >> 13) << 11) | (u & (SEG - 1))
      iblk_v[sl] = ((i >> 13) << 11) | (i & (SEG - 1))

    def fire(c):
      b = c % 2
      sl = pl.ds(c * CH, CH)
      return (
          pltpu.async_copy(gu_hbm.at[ublk_v.at[sl]], guB.at[b], sem_u[b]),
          pltpu.async_copy(gi_hbm.at[iblk_v.at[sl]], giB.at[b], sem_i[b]),
      )

    pending = fire(0)
    for c in range(NCH):
      nxt = fire(c + 1) if c + 1 < NCH else None
      for cp_ in pending:
        cp_.wait()
      b = c % 2

      @pl.loop(0, CH // L)
      def _(gg, c=c, b=b):
        j0 = c * CH + gg * L
        lrow = lax.iota(jnp.int32, L) + gg * L
        u16 = uidx_v[pl.ds(j0, L)]
        i16 = iidx_v[pl.ds(j0, L)]
        cbu = ((u16 >> 11) & 3) << 5
        cbi = ((i16 >> 11) & 3) << 5
        jflat = (lrow + c * CH) << 5
        acc = jnp.zeros((L,), jnp.float32)
        for kk in range(K):
          uk = plsc.load_gather(guB.at[b], [lrow, cbu + kk])
          ik = plsc.load_gather(giB.at[b], [lrow, cbi + kk])
          acc = acc + uk * ik
          plsc.store_scatter(gu_v, [jflat + kk], uk)
          plsc.store_scatter(gi_v, [jflat + kk], ik)
        xui_v[pl.ds(j0, L)] = acc

      pending = nxt

    pltpu.sync_copy(xui_v, xui_hbm.at[pl.ds(base, BPW)])
    pltpu.sync_copy(gu_v, guo_hbm.at[pl.ds(base * K, BPW * K)])
    pltpu.sync_copy(gi_v, gio_hbm.at[pl.ds(base * K, BPW * K)])

  return k(users, items, gu_tab, gi_tab)


def _transpose_block(xt_ref, out_ref):
  x = xt_ref[...]
  out_ref[...] = jnp.concatenate(
      [x[:, r * SEG:(r + 1) * SEG].T for r in range(4)], axis=1)


def _blocked(tab_t):
  return pl.pallas_call(
      _transpose_block,
      grid=(NBLK,),
      in_specs=[pl.BlockSpec((K, CW), lambda c: (0, c))],
      out_specs=pl.BlockSpec((SEG, KB), lambda c: (c, 0)),
      out_shape=jax.ShapeDtypeStruct((VB, KB), jnp.float32),
  )(tab_t)


def kernel(users, items, Gu, Gi):
  users = users.astype(jnp.int32)
  items = items.astype(jnp.int32)
  xui, guo, gio = _bprmf_sc(users, items, _blocked(Gu.T), _blocked(Gi.T))
  return (xui, guo.reshape(B, K), gio.reshape(B, K))
